# Initial kernel scaffold; baseline (speedup 1.0000x reference)
import jax
import jax.numpy as jnp
from jax.experimental import pallas as pl


def kernel(x, last_update, edge_index, t, msg, W1_rel, b1_rel, W1_root, bn_gamma, bn_beta, bn_mean, bn_var, W2_rel, b2_rel, W2_root):
    raise NotImplementedError("write your pallas kernel here")


# speedup vs baseline: 3.2489x; 3.2489x over previous
import functools

import jax
import jax.numpy as jnp
from jax import lax
from jax.experimental import pallas as pl
from jax.experimental.pallas import tpu as pltpu
from jax.experimental.pallas import tpu_sc as plsc

N_SC = 2
N_TILE = 16
CHUNK = 128
EPS = 1e-5


def _segment_sum_partials(n_nodes, d, n_pad, chunks):
    rows_per_tile = n_pad // N_TILE
    mesh = plsc.VectorSubcoreMesh(core_axis_name="c", subcore_axis_name="s")

    @functools.partial(
        pl.kernel,
        mesh=mesh,
        out_type=jax.ShapeDtypeStruct((N_SC, n_pad, d), jnp.float32),
        scratch_types=[
            pltpu.VMEM((chunks, CHUNK), jnp.int32),
            pltpu.VMEM((chunks, CHUNK), jnp.int32),
            pltpu.VMEM((CHUNK, d), jnp.float32),
            pltpu.VMEM_SHARED((n_pad, d), jnp.float32),
            pltpu.SemaphoreType.DMA,
            pltpu.SemaphoreType.DMA,
        ],
    )
    def seg(x_hbm, src_hbm, dst_hbm, out_hbm, src_v, dst_v, buf, acc, sem0, sem1):
        c = lax.axis_index("c")
        s = lax.axis_index("s")
        wid = c * N_TILE + s

        cp_s = pltpu.async_copy(src_hbm.at[wid], src_v, sem0)
        cp_d = pltpu.async_copy(dst_hbm.at[wid], dst_v, sem1)

        @pl.loop(0, CHUNK)
        def _(i):
            @pl.loop(0, d, step=16)
            def _(j):
                buf[i, pl.ds(j, 16)] = jnp.zeros((16,), jnp.float32)

        cp_s.wait()
        cp_d.wait()

        @pl.loop(0, rows_per_tile, step=CHUNK)
        def _(r):
            pltpu.sync_copy(buf, acc.at[pl.ds(s * rows_per_tile + r, CHUNK)])

        plsc.subcore_barrier()

        @pl.loop(0, chunks)
        def _(j):
            pltpu.sync_copy(x_hbm.at[src_v.at[j]], buf)
            pltpu.sync_copy(buf, acc.at[dst_v.at[j]], add=True)

        plsc.subcore_barrier()

        pltpu.sync_copy(
            acc.at[pl.ds(s * rows_per_tile, rows_per_tile)],
            out_hbm.at[c, pl.ds(s * rows_per_tile, rows_per_tile)],
        )

    return seg


def _dense(p, xin, w_rel_t, w_root_t, scale, shift, relu):
    n, d = xin.shape
    n_pad = p.shape[1]
    blk = 2048
    grid = (n_pad // blk,)

    def body(p_ref, x_ref, wr_ref, wo_ref, sc_ref, sh_ref, o_ref):
        agg = (p_ref[0] + p_ref[1]).astype(jnp.bfloat16)
        acc = jnp.dot(agg, wr_ref[...].astype(jnp.bfloat16),
                      preferred_element_type=jnp.float32)
        acc += jnp.dot(x_ref[...].astype(jnp.bfloat16),
                       wo_ref[...].astype(jnp.bfloat16),
                       preferred_element_type=jnp.float32)
        h = acc * sc_ref[...] + sh_ref[...]
        if relu:
            h = jnp.maximum(h, 0.0)
        o_ref[...] = h

    return pl.pallas_call(
        body,
        grid=grid,
        in_specs=[
            pl.BlockSpec((2, blk, d), lambda i: (0, i, 0)),
            pl.BlockSpec((blk, d), lambda i: (i, 0)),
            pl.BlockSpec((d, d), lambda i: (0, 0)),
            pl.BlockSpec((d, d), lambda i: (0, 0)),
            pl.BlockSpec((1, d), lambda i: (0, 0)),
            pl.BlockSpec((1, d), lambda i: (0, 0)),
        ],
        out_specs=pl.BlockSpec((blk, d), lambda i: (i, 0)),
        out_shape=jax.ShapeDtypeStruct((n, d), jnp.float32),
    )(p, xin, w_rel_t, w_root_t, scale, shift)


def kernel(x, last_update, edge_index, t, msg, W1_rel, b1_rel, W1_root,
           bn_gamma, bn_beta, bn_mean, bn_var, W2_rel, b2_rel, W2_root):
    n, d = x.shape
    e = edge_index.shape[1]
    n_tiles = N_SC * N_TILE

    n_pad = (n // (N_TILE * CHUNK) + 1) * (N_TILE * CHUNK)

    chunks = -(-e // (n_tiles * CHUNK))
    chunks += chunks % 2
    e_pad = n_tiles * chunks * CHUNK
    src = jnp.concatenate(
        [edge_index[0], jnp.zeros((e_pad - e,), jnp.int32)]
    ).reshape(n_tiles, chunks, CHUNK)
    dst = jnp.concatenate(
        [edge_index[1], jnp.full((e_pad - e,), n, jnp.int32)]
    ).reshape(n_tiles, chunks, CHUNK)

    seg = _segment_sum_partials(n, d, n_pad, chunks)

    s1 = bn_gamma * lax.rsqrt(bn_var + EPS)
    sh1 = (b1_rel - bn_mean) * s1 + bn_beta
    ones = jnp.ones((d,), jnp.float32)

    p1 = seg(x, src, dst)
    h = _dense(p1, x, W1_rel.T, W1_root.T,
               s1.reshape(1, d), sh1.reshape(1, d), relu=True)
    p2 = seg(h, src, dst)
    out = _dense(p2, h, W2_rel.T, W2_root.T,
                 ones.reshape(1, d), b2_rel.reshape(1, d), relu=False)
    return out


# speedup vs baseline: 3.4061x; 1.0484x over previous
import functools

import jax
import jax.numpy as jnp
from jax import lax
from jax.experimental import pallas as pl
from jax.experimental.pallas import tpu as pltpu
from jax.experimental.pallas import tpu_sc as plsc

N_SC = 2
N_TILE = 16
CHUNK = 64
EPS = 1e-5


def _segment_sum_partials(n_nodes, d, n_pad, chunks):
    rows_per_tile = n_pad // N_TILE
    mesh = plsc.VectorSubcoreMesh(core_axis_name="c", subcore_axis_name="s")

    @functools.partial(
        pl.kernel,
        mesh=mesh,
        out_type=jax.ShapeDtypeStruct((N_SC, n_pad, d), jnp.float32),
        scratch_types=[
            pltpu.VMEM((chunks * CHUNK,), jnp.int32),
            pltpu.VMEM((4, CHUNK), jnp.int32),
            pltpu.VMEM((4, CHUNK), jnp.int32),
            pltpu.VMEM((4, CHUNK, d), jnp.float32),
            pltpu.VMEM_SHARED((n_pad, d), jnp.float32),
            pltpu.SemaphoreType.DMA,
            pltpu.SemaphoreType.DMA,
            pltpu.SemaphoreType.DMA,
            pltpu.SemaphoreType.DMA,
        ],
    )
    def seg(x_hbm, pk_hbm, out_hbm, pk_v, sidx, didx, buf, acc,
            semg_a, semg_b, sems_a, sems_b):
        c = lax.axis_index("c")
        s = lax.axis_index("s")
        wid = c * N_TILE + s

        cp = pltpu.async_copy(pk_hbm.at[wid], pk_v, semg_a)

        @pl.loop(0, CHUNK)
        def _(i):
            @pl.loop(0, d, step=16)
            def _(j):
                buf[0, i, pl.ds(j, 16)] = jnp.zeros((16,), jnp.float32)

        cp.wait()

        base = s * rows_per_tile
        whole = rows_per_tile // CHUNK * CHUNK

        @pl.loop(0, whole, step=CHUNK)
        def _(r):
            pltpu.sync_copy(buf.at[0], acc.at[pl.ds(base + r, CHUNK)])

        if rows_per_tile > whole:
            rem = rows_per_tile - whole
            pltpu.sync_copy(buf.at[0, pl.ds(0, rem)],
                            acc.at[pl.ds(base + whole, rem)])

        plsc.subcore_barrier()

        def unpack(idx, b):
            @pl.loop(0, CHUNK, step=16)
            def _(k):
                v = pk_v[pl.ds(idx * CHUNK + k, 16)]
                sidx[b, pl.ds(k, 16)] = v & 0xFFFF
                didx[b, pl.ds(k, 16)] = v >> 16

        def gather(b, sem):
            pltpu.async_copy(x_hbm.at[sidx.at[b]], buf.at[b], sem)

        def gather_wait(b, sem):
            pltpu.make_async_copy(x_hbm.at[sidx.at[b]], buf.at[b], sem).wait()

        def scat(b, sem):
            pltpu.async_copy(buf.at[b], acc.at[didx.at[b]], sem, add=True)

        def scat_wait(b, sem):
            pltpu.make_async_copy(buf.at[b], acc.at[didx.at[b]], sem).wait()

        unpack(0, 0)
        unpack(1, 1)
        gather(0, semg_a)
        gather(1, semg_a)

        @pl.loop(0, chunks, step=4)
        def _(j):
            gather_wait(0, semg_a)
            gather_wait(1, semg_a)
            scat(0, sems_a)
            scat(1, sems_a)

            @pl.when(j > 0)
            def _():
                scat_wait(2, sems_b)
                scat_wait(3, sems_b)

            unpack(j + 2, 2)
            unpack(j + 3, 3)
            gather(2, semg_b)
            gather(3, semg_b)
            gather_wait(2, semg_b)
            gather_wait(3, semg_b)
            scat(2, sems_b)
            scat(3, sems_b)
            scat_wait(0, sems_a)
            scat_wait(1, sems_a)

            @pl.when(j + 4 < chunks)
            def _():
                unpack(j + 4, 0)
                unpack(j + 5, 1)
                gather(0, semg_a)
                gather(1, semg_a)

        scat_wait(2, sems_b)
        scat_wait(3, sems_b)

        plsc.subcore_barrier()

        pltpu.sync_copy(
            acc.at[pl.ds(base, rows_per_tile)],
            out_hbm.at[c, pl.ds(base, rows_per_tile)],
        )

    return seg


def _dense(p, xin, w_rel_t, w_root_t, scale, shift, relu):
    n, d = xin.shape
    n_pad = p.shape[1]
    blk = 2048
    grid = (-(-n_pad // blk),)

    def body(p_ref, x_ref, wr_ref, wo_ref, sc_ref, sh_ref, o_ref):
        agg = (p_ref[0] + p_ref[1]).astype(jnp.bfloat16)
        acc = jnp.dot(agg, wr_ref[...].astype(jnp.bfloat16),
                      preferred_element_type=jnp.float32)
        acc += jnp.dot(x_ref[...].astype(jnp.bfloat16),
                       wo_ref[...].astype(jnp.bfloat16),
                       preferred_element_type=jnp.float32)
        h = acc * sc_ref[...] + sh_ref[...]
        if relu:
            h = jnp.maximum(h, 0.0)
        o_ref[...] = h

    return pl.pallas_call(
        body,
        grid=grid,
        in_specs=[
            pl.BlockSpec((2, blk, d), lambda i: (0, i, 0)),
            pl.BlockSpec((blk, d), lambda i: (i, 0)),
            pl.BlockSpec((d, d), lambda i: (0, 0)),
            pl.BlockSpec((d, d), lambda i: (0, 0)),
            pl.BlockSpec((1, d), lambda i: (0, 0)),
            pl.BlockSpec((1, d), lambda i: (0, 0)),
        ],
        out_specs=pl.BlockSpec((blk, d), lambda i: (i, 0)),
        out_shape=jax.ShapeDtypeStruct((n, d), jnp.float32),
    )(p, xin, w_rel_t, w_root_t, scale, shift)


def kernel(x, last_update, edge_index, t, msg, W1_rel, b1_rel, W1_root,
           bn_gamma, bn_beta, bn_mean, bn_var, W2_rel, b2_rel, W2_root):
    n, d = x.shape
    e = edge_index.shape[1]
    n_tiles = N_SC * N_TILE

    n_pad = (n // (N_TILE * 8) + 1) * (N_TILE * 8)

    chunks = -(-e // (n_tiles * CHUNK))
    chunks = -(-chunks // 4) * 4
    e_pad = n_tiles * chunks * CHUNK
    packed = edge_index[0] | (edge_index[1] << 16)
    packed = jnp.concatenate(
        [packed, jnp.full((e_pad - e,), n << 16, jnp.int32)]
    ).reshape(n_tiles, chunks * CHUNK)

    seg = _segment_sum_partials(n, d, n_pad, chunks)

    s1 = bn_gamma * lax.rsqrt(bn_var + EPS)
    sh1 = (b1_rel - bn_mean) * s1 + bn_beta
    ones = jnp.ones((d,), jnp.float32)

    p1 = seg(x, packed)
    h = _dense(p1, x, W1_rel.T, W1_root.T,
               s1.reshape(1, d), sh1.reshape(1, d), relu=True)
    p2 = seg(h, packed)
    out = _dense(p2, h, W2_rel.T, W2_root.T,
                 ones.reshape(1, d), b2_rel.reshape(1, d), relu=False)
    return out


# speedup vs baseline: 6.2169x; 1.8252x over previous
import functools

import jax
import jax.numpy as jnp
from jax import lax
from jax.experimental import pallas as pl
from jax.experimental.pallas import tpu as pltpu
from jax.experimental.pallas import tpu_sc as plsc

N_SC = 2
N_TILE = 16
CHUNK = 64
EPS = 1e-5

SC0_CHUNKS = 244
SC1_CHUNKS = 72


def _segment_sum_partials(n_nodes, d, n_pad):
    rows_per_tile = n_pad // N_TILE
    mesh = plsc.VectorSubcoreMesh(core_axis_name="c", subcore_axis_name="s")

    @functools.partial(
        pl.kernel,
        mesh=mesh,
        out_type=jax.ShapeDtypeStruct((N_SC, n_pad, d), jnp.float32),
        scratch_types=[
            pltpu.VMEM((SC0_CHUNKS * CHUNK,), jnp.int32),
            pltpu.VMEM((8, CHUNK), jnp.int32),
            pltpu.VMEM((4, CHUNK, d), jnp.float32),
            pltpu.VMEM_SHARED((n_pad, d), jnp.float32),
            pltpu.SemaphoreType.DMA,
            pltpu.SemaphoreType.DMA,
            pltpu.SemaphoreType.DMA,
            pltpu.SemaphoreType.DMA,
        ],
    )
    def seg(x_hbm, pk_hbm, out_hbm, pk_v, ring, buf, acc,
            semg_a, semg_b, sems_a, sems_b):
        c = lax.axis_index("c")
        s = lax.axis_index("s")
        wid = c * N_TILE + s
        nchunks = jnp.where(c == 0, SC0_CHUNKS, SC1_CHUNKS)

        cp = pltpu.async_copy(pk_hbm.at[wid], pk_v, semg_a)

        @pl.loop(0, CHUNK)
        def _(i):
            @pl.loop(0, d, step=16)
            def _(j):
                buf[0, i, pl.ds(j, 16)] = jnp.zeros((16,), jnp.float32)

        cp.wait()

        base = s * rows_per_tile
        whole = rows_per_tile // CHUNK * CHUNK

        @pl.loop(0, whole, step=CHUNK)
        def _(r):
            pltpu.sync_copy(buf.at[0], acc.at[pl.ds(base + r, CHUNK)])

        if rows_per_tile > whole:
            rem = rows_per_tile - whole
            pltpu.sync_copy(buf.at[0, pl.ds(0, rem)],
                            acc.at[pl.ds(base + whole, rem)])

        plsc.subcore_barrier()

        def unpack(idx, b):
            @pl.loop(0, CHUNK, step=16)
            def _(k):
                v = pk_v[pl.ds(idx * CHUNK + k, 16)]
                ring[b, pl.ds(k, 16)] = v & 0xFFFF
                ring[4 + b, pl.ds(k, 16)] = v >> 16

        def gather(b, sem):
            pltpu.async_copy(x_hbm.at[ring.at[b]], buf.at[b], sem)

        def gather_wait(b, sem):
            pltpu.make_async_copy(x_hbm.at[ring.at[b]], buf.at[b], sem).wait()

        def scat(b, sem):
            pltpu.async_copy(buf.at[b], acc.at[ring.at[4 + b]], sem, add=True)

        def scat_wait(b, sem):
            pltpu.make_async_copy(buf.at[b], acc.at[ring.at[4 + b]], sem).wait()

        unpack(0, 0)
        unpack(1, 1)
        gather(0, semg_a)
        gather(1, semg_a)

        @pl.loop(0, nchunks, step=4)
        def _(j):
            gather_wait(0, semg_a)
            gather_wait(1, semg_a)
            scat(0, sems_a)
            scat(1, sems_a)

            @pl.when(j > 0)
            def _():
                scat_wait(2, sems_b)
                scat_wait(3, sems_b)

            unpack(j + 2, 2)
            unpack(j + 3, 3)
            gather(2, semg_b)
            gather(3, semg_b)
            gather_wait(2, semg_b)
            gather_wait(3, semg_b)
            scat(2, sems_b)
            scat(3, sems_b)
            scat_wait(0, sems_a)
            scat_wait(1, sems_a)

            @pl.when(j + 4 < nchunks)
            def _():
                unpack(j + 4, 0)
                unpack(j + 5, 1)
                gather(0, semg_a)
                gather(1, semg_a)

        scat_wait(2, sems_b)
        scat_wait(3, sems_b)

        plsc.subcore_barrier()

        pltpu.sync_copy(
            acc.at[pl.ds(base, rows_per_tile)],
            out_hbm.at[c, pl.ds(base, rows_per_tile)],
        )

    return seg


def _dense(p, xin, w_rel_t, w_root_t, scale, shift, relu):
    n, d = xin.shape
    n_pad = p.shape[1]
    blk = 2048
    grid = (-(-n_pad // blk),)

    def body(p_ref, x_ref, wr_ref, wo_ref, sc_ref, sh_ref, o_ref):
        agg = (p_ref[0] + p_ref[1]).astype(jnp.bfloat16)
        acc = jnp.dot(agg, wr_ref[...].astype(jnp.bfloat16),
                      preferred_element_type=jnp.float32)
        acc += jnp.dot(x_ref[...].astype(jnp.bfloat16),
                       wo_ref[...].astype(jnp.bfloat16),
                       preferred_element_type=jnp.float32)
        h = acc * sc_ref[...] + sh_ref[...]
        if relu:
            h = jnp.maximum(h, 0.0)
        o_ref[...] = h

    return pl.pallas_call(
        body,
        grid=grid,
        in_specs=[
            pl.BlockSpec((2, blk, d), lambda i: (0, i, 0)),
            pl.BlockSpec((blk, d), lambda i: (i, 0)),
            pl.BlockSpec((d, d), lambda i: (0, 0)),
            pl.BlockSpec((d, d), lambda i: (0, 0)),
            pl.BlockSpec((1, d), lambda i: (0, 0)),
            pl.BlockSpec((1, d), lambda i: (0, 0)),
        ],
        out_specs=pl.BlockSpec((blk, d), lambda i: (i, 0)),
        out_shape=jax.ShapeDtypeStruct((n, d), jnp.float32),
    )(p, xin, w_rel_t, w_root_t, scale, shift)


def kernel(x, last_update, edge_index, t, msg, W1_rel, b1_rel, W1_root,
           bn_gamma, bn_beta, bn_mean, bn_var, W2_rel, b2_rel, W2_root):
    n, d = x.shape
    e = edge_index.shape[1]

    n_pad = (n // (N_TILE * 8) + 1) * (N_TILE * 8)

    e0 = N_TILE * SC0_CHUNKS * CHUNK
    e1 = N_TILE * SC1_CHUNKS * CHUNK
    assert e0 + e1 >= e
    pad_val = jnp.int32(n << 16)
    packed = edge_index[0] | (edge_index[1] << 16)
    packed = jnp.concatenate(
        [packed, jnp.full((e0 + e1 - e,), pad_val, jnp.int32)])
    part0 = packed[:e0].reshape(N_TILE, SC0_CHUNKS * CHUNK)
    part1 = packed[e0:].reshape(N_TILE, SC1_CHUNKS * CHUNK)
    part1 = jnp.pad(part1, ((0, 0), (0, (SC0_CHUNKS - SC1_CHUNKS) * CHUNK)),
                    constant_values=pad_val)
    slab = jnp.concatenate([part0, part1], axis=0)

    seg = _segment_sum_partials(n, d, n_pad)

    s1 = bn_gamma * lax.rsqrt(bn_var + EPS)
    sh1 = (b1_rel - bn_mean) * s1 + bn_beta
    ones = jnp.ones((d,), jnp.float32)

    p1 = seg(x, slab)
    h = _dense(p1, x, W1_rel.T, W1_root.T,
               s1.reshape(1, d), sh1.reshape(1, d), relu=True)
    p2 = seg(h, slab)
    out = _dense(p2, h, W2_rel.T, W2_root.T,
                 ones.reshape(1, d), b2_rel.reshape(1, d), relu=False)
    return out


# speedup vs baseline: 6.3541x; 1.0221x over previous
import functools

import jax
import jax.numpy as jnp
from jax import lax
from jax.experimental import pallas as pl
from jax.experimental.pallas import tpu as pltpu
from jax.experimental.pallas import tpu_sc as plsc

N_SC = 2
N_TILE = 16
CHUNK = 64
EPS = 1e-5

SC0_CHUNKS = 252
SC1_CHUNKS = 64


def _segment_sum_partials(n_nodes, d, n_pad):
    rows_per_tile = n_pad // N_TILE
    mesh = plsc.VectorSubcoreMesh(core_axis_name="c", subcore_axis_name="s")

    @functools.partial(
        pl.kernel,
        mesh=mesh,
        out_type=jax.ShapeDtypeStruct((N_SC, n_pad, d), jnp.float32),
        scratch_types=[
            pltpu.VMEM((SC0_CHUNKS * CHUNK,), jnp.int32),
            pltpu.VMEM((8, CHUNK), jnp.int32),
            pltpu.VMEM((4, CHUNK, d), jnp.float32),
            pltpu.VMEM_SHARED((n_pad, d), jnp.float32),
            pltpu.SemaphoreType.DMA,
            pltpu.SemaphoreType.DMA,
            pltpu.SemaphoreType.DMA,
            pltpu.SemaphoreType.DMA,
        ],
    )
    def seg(x_hbm, pk_hbm, out_hbm, pk_v, ring, buf, acc,
            semg_a, semg_b, sems_a, sems_b):
        c = lax.axis_index("c")
        s = lax.axis_index("s")
        wid = c * N_TILE + s
        nchunks = jnp.where(c == 0, SC0_CHUNKS, SC1_CHUNKS)

        cp = pltpu.async_copy(pk_hbm.at[wid], pk_v, semg_a)

        @pl.loop(0, CHUNK)
        def _(i):
            @pl.loop(0, d, step=16)
            def _(j):
                buf[0, i, pl.ds(j, 16)] = jnp.zeros((16,), jnp.float32)

        cp.wait()

        base = s * rows_per_tile
        whole = rows_per_tile // CHUNK * CHUNK

        @pl.loop(0, whole, step=CHUNK)
        def _(r):
            pltpu.sync_copy(buf.at[0], acc.at[pl.ds(base + r, CHUNK)])

        if rows_per_tile > whole:
            rem = rows_per_tile - whole
            pltpu.sync_copy(buf.at[0, pl.ds(0, rem)],
                            acc.at[pl.ds(base + whole, rem)])

        plsc.subcore_barrier()

        def unpack(idx, b):
            @pl.loop(0, CHUNK, step=16)
            def _(k):
                v = pk_v[pl.ds(idx * CHUNK + k, 16)]
                ring[b, pl.ds(k, 16)] = v & 0xFFFF
                ring[4 + b, pl.ds(k, 16)] = v >> 16

        def gather(b, sem):
            pltpu.async_copy(x_hbm.at[ring.at[b]], buf.at[b], sem)

        def gather_wait(b, sem):
            pltpu.make_async_copy(x_hbm.at[ring.at[b]], buf.at[b], sem).wait()

        def scat(b, sem):
            pltpu.async_copy(buf.at[b], acc.at[ring.at[4 + b]], sem, add=True)

        def scat_wait(b, sem):
            pltpu.make_async_copy(buf.at[b], acc.at[ring.at[4 + b]], sem).wait()

        unpack(0, 0)
        unpack(1, 1)
        gather(0, semg_a)
        gather(1, semg_a)

        @pl.loop(0, nchunks, step=4)
        def _(j):
            gather_wait(0, semg_a)
            gather_wait(1, semg_a)
            scat(0, sems_a)
            scat(1, sems_a)

            @pl.when(j > 0)
            def _():
                scat_wait(2, sems_b)
                scat_wait(3, sems_b)

            unpack(j + 2, 2)
            unpack(j + 3, 3)
            gather(2, semg_b)
            gather(3, semg_b)
            gather_wait(2, semg_b)
            gather_wait(3, semg_b)
            scat(2, sems_b)
            scat(3, sems_b)
            scat_wait(0, sems_a)
            scat_wait(1, sems_a)

            @pl.when(j + 4 < nchunks)
            def _():
                unpack(j + 4, 0)
                unpack(j + 5, 1)
                gather(0, semg_a)
                gather(1, semg_a)

        scat_wait(2, sems_b)
        scat_wait(3, sems_b)

        plsc.subcore_barrier()

        pltpu.sync_copy(
            acc.at[pl.ds(base, rows_per_tile)],
            out_hbm.at[c, pl.ds(base, rows_per_tile)],
        )

    return seg


def _dense(p, xin, w_rel_t, w_root_t, scale, shift, relu):
    n, d = xin.shape
    n_pad = p.shape[1]
    blk = 2048
    grid = (-(-n_pad // blk),)

    def body(p_ref, x_ref, wr_ref, wo_ref, sc_ref, sh_ref, o_ref):
        agg = (p_ref[0] + p_ref[1]).astype(jnp.bfloat16)
        acc = jnp.dot(agg, wr_ref[...].astype(jnp.bfloat16),
                      preferred_element_type=jnp.float32)
        acc += jnp.dot(x_ref[...].astype(jnp.bfloat16),
                       wo_ref[...].astype(jnp.bfloat16),
                       preferred_element_type=jnp.float32)
        h = acc * sc_ref[...] + sh_ref[...]
        if relu:
            h = jnp.maximum(h, 0.0)
        o_ref[...] = h

    return pl.pallas_call(
        body,
        grid=grid,
        in_specs=[
            pl.BlockSpec((2, blk, d), lambda i: (0, i, 0)),
            pl.BlockSpec((blk, d), lambda i: (i, 0)),
            pl.BlockSpec((d, d), lambda i: (0, 0)),
            pl.BlockSpec((d, d), lambda i: (0, 0)),
            pl.BlockSpec((1, d), lambda i: (0, 0)),
            pl.BlockSpec((1, d), lambda i: (0, 0)),
        ],
        out_specs=pl.BlockSpec((blk, d), lambda i: (i, 0)),
        out_shape=jax.ShapeDtypeStruct((n, d), jnp.float32),
    )(p, xin, w_rel_t, w_root_t, scale, shift)


def kernel(x, last_update, edge_index, t, msg, W1_rel, b1_rel, W1_root,
           bn_gamma, bn_beta, bn_mean, bn_var, W2_rel, b2_rel, W2_root):
    n, d = x.shape
    e = edge_index.shape[1]

    n_pad = (n // (N_TILE * 8) + 1) * (N_TILE * 8)

    e0 = N_TILE * SC0_CHUNKS * CHUNK
    e1 = N_TILE * SC1_CHUNKS * CHUNK
    assert e0 + e1 >= e
    pad_val = jnp.int32(n << 16)
    packed = edge_index[0] | (edge_index[1] << 16)
    packed = jnp.concatenate(
        [packed, jnp.full((e0 + e1 - e,), pad_val, jnp.int32)])
    part0 = packed[:e0].reshape(N_TILE, SC0_CHUNKS * CHUNK)
    part1 = packed[e0:].reshape(N_TILE, SC1_CHUNKS * CHUNK)
    part1 = jnp.pad(part1, ((0, 0), (0, (SC0_CHUNKS - SC1_CHUNKS) * CHUNK)),
                    constant_values=pad_val)
    slab = jnp.concatenate([part0, part1], axis=0)

    seg = _segment_sum_partials(n, d, n_pad)

    s1 = bn_gamma * lax.rsqrt(bn_var + EPS)
    sh1 = (b1_rel - bn_mean) * s1 + bn_beta
    ones = jnp.ones((d,), jnp.float32)

    p1 = seg(x, slab)
    h = _dense(p1, x, W1_rel.T, W1_root.T,
               s1.reshape(1, d), sh1.reshape(1, d), relu=True)
    p2 = seg(h, slab)
    out = _dense(p2, h, W2_rel.T, W2_root.T,
                 ones.reshape(1, d), b2_rel.reshape(1, d), relu=False)
    return out


# speedup vs baseline: 6.8347x; 1.0757x over previous
import functools

import jax
import jax.numpy as jnp
from jax import lax
from jax.experimental import pallas as pl
from jax.experimental.pallas import tpu as pltpu
from jax.experimental.pallas import tpu_sc as plsc

N_SC = 2
N_TILE = 16
CHUNK = 64
EPS = 1e-5

SC0_CHUNKS = 252
SC1_CHUNKS = 64


def _segment_sum_partials(n_nodes, d, n_pad):
    rows_per_tile = n_pad // N_TILE
    mesh = plsc.VectorSubcoreMesh(core_axis_name="c", subcore_axis_name="s")

    @functools.partial(
        pl.kernel,
        mesh=mesh,
        out_type=jax.ShapeDtypeStruct((N_SC, n_pad, d), jnp.float32),
        scratch_types=[
            pltpu.VMEM((SC0_CHUNKS * CHUNK,), jnp.int32),
            pltpu.VMEM((8, CHUNK), jnp.int32),
            pltpu.VMEM((4, CHUNK, d), jnp.float32),
            pltpu.VMEM_SHARED((n_pad, d), jnp.float32),
            pltpu.SemaphoreType.DMA,
            pltpu.SemaphoreType.DMA,
            pltpu.SemaphoreType.DMA,
            pltpu.SemaphoreType.DMA,
        ],
    )
    def seg(x_hbm, pk_hbm, out_hbm, pk_v, ring, buf, acc,
            semg_a, semg_b, sems_a, sems_b):
        c = lax.axis_index("c")
        s = lax.axis_index("s")
        nchunks = jnp.where(c == 0, SC0_CHUNKS, SC1_CHUNKS)
        off = jnp.where(c == 0, s * (SC0_CHUNKS * CHUNK),
                        N_TILE * SC0_CHUNKS * CHUNK + s * (SC1_CHUNKS * CHUNK))

        cp = pltpu.async_copy(pk_hbm.at[pl.ds(off, SC0_CHUNKS * CHUNK)],
                              pk_v, semg_a)

        @pl.loop(0, CHUNK)
        def _(i):
            @pl.loop(0, d, step=16)
            def _(j):
                buf[0, i, pl.ds(j, 16)] = jnp.zeros((16,), jnp.float32)

        cp.wait()

        base = s * rows_per_tile
        whole = rows_per_tile // CHUNK * CHUNK

        @pl.loop(0, whole, step=CHUNK)
        def _(r):
            pltpu.sync_copy(buf.at[0], acc.at[pl.ds(base + r, CHUNK)])

        if rows_per_tile > whole:
            rem = rows_per_tile - whole
            pltpu.sync_copy(buf.at[0, pl.ds(0, rem)],
                            acc.at[pl.ds(base + whole, rem)])

        plsc.subcore_barrier()

        def unpack(idx, b):
            @pl.loop(0, CHUNK, step=16)
            def _(k):
                v = pk_v[pl.ds(idx * CHUNK + k, 16)]
                ring[b, pl.ds(k, 16)] = v & 0xFFFF
                ring[4 + b, pl.ds(k, 16)] = v >> 16

        def gather(b, sem):
            pltpu.async_copy(x_hbm.at[ring.at[b]], buf.at[b], sem)

        def gather_wait(b, sem):
            pltpu.make_async_copy(x_hbm.at[ring.at[b]], buf.at[b], sem).wait()

        def scat(b, sem):
            pltpu.async_copy(buf.at[b], acc.at[ring.at[4 + b]], sem, add=True)

        def scat_wait(b, sem):
            pltpu.make_async_copy(buf.at[b], acc.at[ring.at[4 + b]], sem).wait()

        unpack(0, 0)
        unpack(1, 1)
        gather(0, semg_a)
        gather(1, semg_a)

        @pl.loop(0, nchunks, step=4)
        def _(j):
            gather_wait(0, semg_a)
            gather_wait(1, semg_a)
            scat(0, sems_a)
            scat(1, sems_a)

            @pl.when(j > 0)
            def _():
                scat_wait(2, sems_b)
                scat_wait(3, sems_b)

            unpack(j + 2, 2)
            unpack(j + 3, 3)
            gather(2, semg_b)
            gather(3, semg_b)
            gather_wait(2, semg_b)
            gather_wait(3, semg_b)
            scat(2, sems_b)
            scat(3, sems_b)
            scat_wait(0, sems_a)
            scat_wait(1, sems_a)

            @pl.when(j + 4 < nchunks)
            def _():
                unpack(j + 4, 0)
                unpack(j + 5, 1)
                gather(0, semg_a)
                gather(1, semg_a)

        scat_wait(2, sems_b)
        scat_wait(3, sems_b)

        plsc.subcore_barrier()

        pltpu.sync_copy(
            acc.at[pl.ds(base, rows_per_tile)],
            out_hbm.at[c, pl.ds(base, rows_per_tile)],
        )

    return seg


def _dense(p, xin, w_rel_t, w_root_t, scale, shift, relu):
    n, d = xin.shape
    n_pad = p.shape[1]
    blk = 2048
    grid = (-(-n_pad // blk),)

    def body(p_ref, x_ref, wr_ref, wo_ref, sc_ref, sh_ref, o_ref):
        agg = (p_ref[0] + p_ref[1]).astype(jnp.bfloat16)
        acc = jnp.dot(agg, wr_ref[...].astype(jnp.bfloat16),
                      preferred_element_type=jnp.float32)
        acc += jnp.dot(x_ref[...].astype(jnp.bfloat16),
                       wo_ref[...].astype(jnp.bfloat16),
                       preferred_element_type=jnp.float32)
        h = acc * sc_ref[...] + sh_ref[...]
        if relu:
            h = jnp.maximum(h, 0.0)
        o_ref[...] = h

    return pl.pallas_call(
        body,
        grid=grid,
        in_specs=[
            pl.BlockSpec((2, blk, d), lambda i: (0, i, 0)),
            pl.BlockSpec((blk, d), lambda i: (i, 0)),
            pl.BlockSpec((d, d), lambda i: (0, 0)),
            pl.BlockSpec((d, d), lambda i: (0, 0)),
            pl.BlockSpec((1, d), lambda i: (0, 0)),
            pl.BlockSpec((1, d), lambda i: (0, 0)),
        ],
        out_specs=pl.BlockSpec((blk, d), lambda i: (i, 0)),
        out_shape=jax.ShapeDtypeStruct((n, d), jnp.float32),
    )(p, xin, w_rel_t, w_root_t, scale, shift)


def kernel(x, last_update, edge_index, t, msg, W1_rel, b1_rel, W1_root,
           bn_gamma, bn_beta, bn_mean, bn_var, W2_rel, b2_rel, W2_root):
    n, d = x.shape
    e = edge_index.shape[1]

    n_pad = (n // (N_TILE * 8) + 1) * (N_TILE * 8)

    e0 = N_TILE * SC0_CHUNKS * CHUNK
    e1 = N_TILE * SC1_CHUNKS * CHUNK
    assert e0 + e1 >= e
    flat_len = e0 + e1 + (SC0_CHUNKS - SC1_CHUNKS) * CHUNK
    pad_val = jnp.int32(n << 16)
    packed = edge_index[0] | (edge_index[1] << 16)
    slab = jnp.concatenate(
        [packed, jnp.full((flat_len - e,), pad_val, jnp.int32)])

    seg = _segment_sum_partials(n, d, n_pad)

    s1 = bn_gamma * lax.rsqrt(bn_var + EPS)
    sh1 = (b1_rel - bn_mean) * s1 + bn_beta
    ones = jnp.ones((d,), jnp.float32)

    p1 = seg(x, slab)
    h = _dense(p1, x, W1_rel.T, W1_root.T,
               s1.reshape(1, d), sh1.reshape(1, d), relu=True)
    p2 = seg(h, slab)
    out = _dense(p2, h, W2_rel.T, W2_root.T,
                 ones.reshape(1, d), b2_rel.reshape(1, d), relu=False)
    return out


# speedup vs baseline: 8.8407x; 1.2935x over previous
import functools

import jax
import jax.numpy as jnp
from jax import lax
from jax.experimental import pallas as pl
from jax.experimental.pallas import tpu as pltpu
from jax.experimental.pallas import tpu_sc as plsc

N_SC = 2
N_TILE = 16
CHUNK = 80
EPS = 1e-5
SC1_SHARE = 0.208


def _segment_sum_partials(n_nodes, d, n_pad, total_chunks):
    n_edges = total_chunks * CHUNK
    rows_per_tile = n_pad // N_TILE
    mesh = plsc.VectorSubcoreMesh(core_axis_name="c", subcore_axis_name="s")

    c1_total = int(round(total_chunks * SC1_SHARE / 8)) * 8
    c0_total = total_chunks - c1_total
    assert c0_total % 8 == 0
    b0 = c0_total // 16 // 8 * 8
    r0 = (c0_total - 16 * b0) // 8
    b1 = c1_total // 16 // 8 * 8
    r1 = (c1_total - 16 * b1) // 8
    assert b1 >= 8 and r0 <= 16 and r1 <= 16

    @functools.partial(
        pl.kernel,
        mesh=mesh,
        out_type=jax.ShapeDtypeStruct((N_SC, n_pad, d), jnp.float32),
        scratch_types=[
            pltpu.VMEM((8 * CHUNK,), jnp.int32),
            pltpu.VMEM((8 * CHUNK,), jnp.int32),
            pltpu.VMEM((4, CHUNK), jnp.int32),
            pltpu.VMEM((4, CHUNK, d), jnp.float32),
            pltpu.VMEM_SHARED((n_pad, d), jnp.float32),
            pltpu.SemaphoreType.DMA,
            pltpu.SemaphoreType.DMA,
            pltpu.SemaphoreType.DMA,
            pltpu.SemaphoreType.DMA,
            pltpu.SemaphoreType.DMA,
            pltpu.SemaphoreType.DMA,
            pltpu.SemaphoreType.DMA,
            pltpu.SemaphoreType.DMA,
        ],
    )
    def seg(x_hbm, ei_hbm, out_hbm, sring, dring, wring, buf, acc,
            semg_a, semg_b, sems_a, sems_b, semi0, semi1, semi2, semi3):
        c = lax.axis_index("c")
        s = lax.axis_index("s")
        is0 = c == 0
        nch = jnp.where(is0, b0 + 8 * (s < r0), b1 + 8 * (s < r1))
        off = jnp.where(is0, b0 * s + 8 * jnp.minimum(s, r0),
                        c0_total + b1 * s + 8 * jnp.minimum(s, r1))
        semi = [semi0, semi1, semi2, semi3]

        def load_idx(gbase, slot):
            e0 = (off + gbase) * CHUNK
            pltpu.async_copy(ei_hbm.at[pl.ds(e0, 2 * CHUNK)],
                             sring.at[pl.ds(slot * 2 * CHUNK, 2 * CHUNK)],
                             semi[slot])
            pltpu.async_copy(ei_hbm.at[pl.ds(n_edges + e0, 2 * CHUNK)],
                             dring.at[pl.ds(slot * 2 * CHUNK, 2 * CHUNK)],
                             semi[slot])

        def wait_idx(gbase, slot):
            e0 = (off + gbase) * CHUNK
            pltpu.make_async_copy(
                ei_hbm.at[pl.ds(e0, 2 * CHUNK)],
                sring.at[pl.ds(slot * 2 * CHUNK, 2 * CHUNK)],
                semi[slot]).wait()
            pltpu.make_async_copy(
                ei_hbm.at[pl.ds(n_edges + e0, 2 * CHUNK)],
                dring.at[pl.ds(slot * 2 * CHUNK, 2 * CHUNK)],
                semi[slot]).wait()

        def dstcopy(slot, row0):
            for q in (0, 1):
                for k in range(0, CHUNK, 16):
                    wring[row0 + q, pl.ds(k, 16)] = dring[
                        pl.ds((2 * slot + q) * CHUNK + k, 16)]

        def gather(slot, dslot0, sem):
            for q in (0, 1):
                pltpu.async_copy(
                    x_hbm.at[sring.at[pl.ds((2 * slot + q) * CHUNK, CHUNK)]],
                    buf.at[dslot0 + q], sem)

        def gather_wait(slot, dslot0, sem):
            for q in (0, 1):
                pltpu.make_async_copy(
                    x_hbm.at[sring.at[pl.ds((2 * slot + q) * CHUNK, CHUNK)]],
                    buf.at[dslot0 + q], sem).wait()

        def scat(dslot0, sem):
            for q in (0, 1):
                pltpu.async_copy(buf.at[dslot0 + q],
                                 acc.at[wring.at[dslot0 + q]], sem, add=True)

        def scat_wait(dslot0, sem):
            for q in (0, 1):
                pltpu.make_async_copy(buf.at[dslot0 + q],
                                      acc.at[wring.at[dslot0 + q]],
                                      sem).wait()

        for g in range(4):
            load_idx(2 * g, g)

        @pl.loop(0, CHUNK)
        def _(i):
            @pl.loop(0, d, step=16)
            def _(j):
                buf[0, i, pl.ds(j, 16)] = jnp.zeros((16,), jnp.float32)

        base = s * rows_per_tile
        whole = rows_per_tile // CHUNK * CHUNK

        @pl.loop(0, whole, step=CHUNK)
        def _(r):
            pltpu.sync_copy(buf.at[0], acc.at[pl.ds(base + r, CHUNK)])

        if rows_per_tile > whole:
            rem = rows_per_tile - whole
            pltpu.sync_copy(buf.at[0, pl.ds(0, rem)],
                            acc.at[pl.ds(base + whole, rem)])

        plsc.subcore_barrier()

        wait_idx(0, 0)
        gather(0, 0, semg_a)

        @pl.loop(0, nch, step=8)
        def _(j):
            gsems = (semg_a, semg_b)
            ssems = (sems_a, sems_b)
            for p in range(4):
                d0 = 2 * (p % 2)
                nd0 = 2 * ((p + 1) % 2)
                gather_wait(p, d0, gsems[p % 2])
                dstcopy(p, d0)
                scat(d0, ssems[p % 2])
                if p == 0:
                    @pl.when(j > 0)
                    def _():
                        scat_wait(2, ssems[1])
                else:
                    scat_wait(nd0, ssems[(p + 1) % 2])

                @pl.when(j + 8 + 2 * p < nch)
                def _():
                    load_idx(j + 8 + 2 * p, p)

                if p < 3:
                    wait_idx(j + 2 * (p + 1), p + 1)
                    gather(p + 1, nd0, gsems[(p + 1) % 2])
                else:
                    @pl.when(j + 8 < nch)
                    def _():
                        wait_idx(j + 8, 0)
                        gather(0, 0, gsems[0])

        scat_wait(2, sems_b)

        plsc.subcore_barrier()

        pltpu.sync_copy(
            acc.at[pl.ds(base, rows_per_tile)],
            out_hbm.at[c, pl.ds(base, rows_per_tile)],
        )

    return seg


def _dense(p, xin, w_rel_t, w_root_t, scale, shift, relu):
    n, d = xin.shape
    n_pad = p.shape[1]
    blk = 2048
    grid = (-(-n_pad // blk),)

    def body(p_ref, x_ref, wr_ref, wo_ref, sc_ref, sh_ref, o_ref):
        agg = (p_ref[0] + p_ref[1]).astype(jnp.bfloat16)
        acc = jnp.dot(agg, wr_ref[...].astype(jnp.bfloat16),
                      preferred_element_type=jnp.float32)
        acc += jnp.dot(x_ref[...].astype(jnp.bfloat16),
                       wo_ref[...].astype(jnp.bfloat16),
                       preferred_element_type=jnp.float32)
        h = acc * sc_ref[...] + sh_ref[...]
        if relu:
            h = jnp.maximum(h, 0.0)
        o_ref[...] = h

    return pl.pallas_call(
        body,
        grid=grid,
        in_specs=[
            pl.BlockSpec((2, blk, d), lambda i: (0, i, 0)),
            pl.BlockSpec((blk, d), lambda i: (i, 0)),
            pl.BlockSpec((d, d), lambda i: (0, 0)),
            pl.BlockSpec((d, d), lambda i: (0, 0)),
            pl.BlockSpec((1, d), lambda i: (0, 0)),
            pl.BlockSpec((1, d), lambda i: (0, 0)),
        ],
        out_specs=pl.BlockSpec((blk, d), lambda i: (i, 0)),
        out_shape=jax.ShapeDtypeStruct((n, d), jnp.float32),
    )(p, xin, w_rel_t, w_root_t, scale, shift)


def kernel(x, last_update, edge_index, t, msg, W1_rel, b1_rel, W1_root,
           bn_gamma, bn_beta, bn_mean, bn_var, W2_rel, b2_rel, W2_root):
    n, d = x.shape
    e = edge_index.shape[1]
    assert e % CHUNK == 0

    n_pad = -(-n // (N_TILE * 8)) * (N_TILE * 8)

    seg = _segment_sum_partials(n, d, n_pad, e // CHUNK)

    s1 = bn_gamma * lax.rsqrt(bn_var + EPS)
    sh1 = (b1_rel - bn_mean) * s1 + bn_beta
    ones = jnp.ones((d,), jnp.float32)

    ei_flat = edge_index.reshape(-1)
    p1 = seg(x, ei_flat)
    h = _dense(p1, x, W1_rel.T, W1_root.T,
               s1.reshape(1, d), sh1.reshape(1, d), relu=True)
    p2 = seg(h, ei_flat)
    out = _dense(p2, h, W2_rel.T, W2_root.T,
                 ones.reshape(1, d), b2_rel.reshape(1, d), relu=False)
    return out


# speedup vs baseline: 11.4404x; 1.2941x over previous
import functools

import jax
import jax.numpy as jnp
from jax import lax
from jax.experimental import pallas as pl
from jax.experimental.pallas import tpu as pltpu
from jax.experimental.pallas import tpu_sc as plsc

N_SC = 2
N_TILE = 16
CHUNK = 80
EPS = 1e-5
SC1_SHARE = 0.442


def _segment_sum_partials(n_nodes, d, n_pad, total_chunks):
    n_edges = total_chunks * CHUNK
    rows_per_tile = n_pad // N_TILE
    mesh = plsc.VectorSubcoreMesh(core_axis_name="c", subcore_axis_name="s")

    c1_total = int(round(total_chunks * SC1_SHARE / 8)) * 8
    c0_total = total_chunks - c1_total
    assert c0_total % 8 == 0
    b0 = c0_total // 16 // 8 * 8
    r0 = (c0_total - 16 * b0) // 8
    b1 = c1_total // 16 // 8 * 8
    r1 = (c1_total - 16 * b1) // 8
    assert b1 >= 8 and r0 <= 16 and r1 <= 16

    @functools.partial(
        pl.kernel,
        mesh=mesh,
        out_type=jax.ShapeDtypeStruct((N_SC, n_pad, d), jnp.float32),
        scratch_types=[
            pltpu.VMEM((8 * CHUNK,), jnp.int32),
            pltpu.VMEM((8 * CHUNK,), jnp.int32),
            pltpu.VMEM((4, CHUNK), jnp.int32),
            pltpu.VMEM((4, CHUNK, d), jnp.float32),
            pltpu.VMEM_SHARED((n_pad, d), jnp.float32),
            pltpu.SemaphoreType.DMA,
            pltpu.SemaphoreType.DMA,
            pltpu.SemaphoreType.DMA,
            pltpu.SemaphoreType.DMA,
            pltpu.SemaphoreType.DMA,
            pltpu.SemaphoreType.DMA,
            pltpu.SemaphoreType.DMA,
            pltpu.SemaphoreType.DMA,
        ],
    )
    def seg(x_hbm, ei_hbm, out_hbm, sring, dring, wring, buf, acc,
            semg_a, semg_b, sems_a, sems_b, semi0, semi1, semi2, semi3):
        c = lax.axis_index("c")
        s = lax.axis_index("s")
        is0 = c == 0
        nch = jnp.where(is0, b0 + 8 * (s < r0), b1 + 8 * (s < r1))
        off = jnp.where(is0, b0 * s + 8 * jnp.minimum(s, r0),
                        c0_total + b1 * s + 8 * jnp.minimum(s, r1))
        semi = [semi0, semi1, semi2, semi3]

        def load_idx(gbase, slot):
            e0 = (off + gbase) * CHUNK
            pltpu.async_copy(ei_hbm.at[pl.ds(e0, 2 * CHUNK)],
                             sring.at[pl.ds(slot * 2 * CHUNK, 2 * CHUNK)],
                             semi[slot])
            pltpu.async_copy(ei_hbm.at[pl.ds(n_edges + e0, 2 * CHUNK)],
                             dring.at[pl.ds(slot * 2 * CHUNK, 2 * CHUNK)],
                             semi[slot])

        def wait_idx(gbase, slot):
            e0 = (off + gbase) * CHUNK
            pltpu.make_async_copy(
                ei_hbm.at[pl.ds(e0, 2 * CHUNK)],
                sring.at[pl.ds(slot * 2 * CHUNK, 2 * CHUNK)],
                semi[slot]).wait()
            pltpu.make_async_copy(
                ei_hbm.at[pl.ds(n_edges + e0, 2 * CHUNK)],
                dring.at[pl.ds(slot * 2 * CHUNK, 2 * CHUNK)],
                semi[slot]).wait()

        def dstcopy(slot, row0):
            for q in (0, 1):
                for k in range(0, CHUNK, 16):
                    wring[row0 + q, pl.ds(k, 16)] = dring[
                        pl.ds((2 * slot + q) * CHUNK + k, 16)]

        def gather(slot, dslot0, sem):
            for q in (0, 1):
                pltpu.async_copy(
                    x_hbm.at[sring.at[pl.ds((2 * slot + q) * CHUNK, CHUNK)]],
                    buf.at[dslot0 + q], sem)

        def gather_wait(slot, dslot0, sem):
            for q in (0, 1):
                pltpu.make_async_copy(
                    x_hbm.at[sring.at[pl.ds((2 * slot + q) * CHUNK, CHUNK)]],
                    buf.at[dslot0 + q], sem).wait()

        def scat(dslot0, sem):
            for q in (0, 1):
                pltpu.async_copy(buf.at[dslot0 + q],
                                 acc.at[wring.at[dslot0 + q]], sem, add=True)

        def scat_wait(dslot0, sem):
            for q in (0, 1):
                pltpu.make_async_copy(buf.at[dslot0 + q],
                                      acc.at[wring.at[dslot0 + q]],
                                      sem).wait()

        for g in range(4):
            load_idx(2 * g, g)

        @pl.loop(0, CHUNK)
        def _(i):
            @pl.loop(0, d, step=16)
            def _(j):
                buf[0, i, pl.ds(j, 16)] = jnp.zeros((16,), jnp.float32)

        base = s * rows_per_tile
        whole = rows_per_tile // CHUNK * CHUNK

        @pl.loop(0, whole, step=CHUNK)
        def _(r):
            pltpu.sync_copy(buf.at[0], acc.at[pl.ds(base + r, CHUNK)])

        if rows_per_tile > whole:
            rem = rows_per_tile - whole
            pltpu.sync_copy(buf.at[0, pl.ds(0, rem)],
                            acc.at[pl.ds(base + whole, rem)])

        plsc.subcore_barrier()

        wait_idx(0, 0)
        gather(0, 0, semg_a)

        @pl.loop(0, nch, step=8)
        def _(j):
            gsems = (semg_a, semg_b)
            ssems = (sems_a, sems_b)
            for p in range(4):
                d0 = 2 * (p % 2)
                nd0 = 2 * ((p + 1) % 2)
                gather_wait(p, d0, gsems[p % 2])
                dstcopy(p, d0)
                scat(d0, ssems[p % 2])
                if p == 0:
                    @pl.when(j > 0)
                    def _():
                        scat_wait(2, ssems[1])
                else:
                    scat_wait(nd0, ssems[(p + 1) % 2])

                @pl.when(j + 8 + 2 * p < nch)
                def _():
                    load_idx(j + 8 + 2 * p, p)

                if p < 3:
                    wait_idx(j + 2 * (p + 1), p + 1)
                    gather(p + 1, nd0, gsems[(p + 1) % 2])
                else:
                    @pl.when(j + 8 < nch)
                    def _():
                        wait_idx(j + 8, 0)
                        gather(0, 0, gsems[0])

        scat_wait(2, sems_b)

        plsc.subcore_barrier()

        pltpu.sync_copy(
            acc.at[pl.ds(base, rows_per_tile)],
            out_hbm.at[c, pl.ds(base, rows_per_tile)],
        )

    return seg


def _dense(p, xin, w_rel_t, w_root_t, scale, shift, relu):
    n, d = xin.shape
    n_pad = p.shape[1]
    blk = 2048
    grid = (-(-n_pad // blk),)

    def body(p_ref, x_ref, wr_ref, wo_ref, sc_ref, sh_ref, o_ref):
        agg = (p_ref[0] + p_ref[1]).astype(jnp.bfloat16)
        acc = jnp.dot(agg, wr_ref[...].astype(jnp.bfloat16),
                      preferred_element_type=jnp.float32)
        acc += jnp.dot(x_ref[...].astype(jnp.bfloat16),
                       wo_ref[...].astype(jnp.bfloat16),
                       preferred_element_type=jnp.float32)
        h = acc * sc_ref[...] + sh_ref[...]
        if relu:
            h = jnp.maximum(h, 0.0)
        o_ref[...] = h

    return pl.pallas_call(
        body,
        grid=grid,
        in_specs=[
            pl.BlockSpec((2, blk, d), lambda i: (0, i, 0)),
            pl.BlockSpec((blk, d), lambda i: (i, 0)),
            pl.BlockSpec((d, d), lambda i: (0, 0)),
            pl.BlockSpec((d, d), lambda i: (0, 0)),
            pl.BlockSpec((1, d), lambda i: (0, 0)),
            pl.BlockSpec((1, d), lambda i: (0, 0)),
        ],
        out_specs=pl.BlockSpec((blk, d), lambda i: (i, 0)),
        out_shape=jax.ShapeDtypeStruct((n, d), jnp.float32),
    )(p, xin, w_rel_t, w_root_t, scale, shift)


def kernel(x, last_update, edge_index, t, msg, W1_rel, b1_rel, W1_root,
           bn_gamma, bn_beta, bn_mean, bn_var, W2_rel, b2_rel, W2_root):
    n, d = x.shape
    e = edge_index.shape[1]
    assert e % CHUNK == 0

    n_pad = -(-n // (N_TILE * 8)) * (N_TILE * 8)

    seg = _segment_sum_partials(n, d, n_pad, e // CHUNK)

    s1 = bn_gamma * lax.rsqrt(bn_var + EPS)
    sh1 = (b1_rel - bn_mean) * s1 + bn_beta
    ones = jnp.ones((d,), jnp.float32)

    ei_flat = edge_index.reshape(-1)
    p1 = seg(x, ei_flat)
    h = _dense(p1, x, W1_rel.T, W1_root.T,
               s1.reshape(1, d), sh1.reshape(1, d), relu=True)
    p2 = seg(h, ei_flat)
    out = _dense(p2, h, W2_rel.T, W2_root.T,
                 ones.reshape(1, d), b2_rel.reshape(1, d), relu=False)
    return out


# speedup vs baseline: 12.2785x; 1.0733x over previous
import functools

import jax
import jax.numpy as jnp
from jax import lax
from jax.experimental import pallas as pl
from jax.experimental.pallas import tpu as pltpu
from jax.experimental.pallas import tpu_sc as plsc

N_SC = 2
N_TILE = 16
CHUNK = 80
EPS = 1e-5
SC1_SHARE = 0.5


def _segment_sum_partials(n_nodes, d, n_pad, total_chunks):
    n_edges = total_chunks * CHUNK
    rows_per_tile = n_pad // N_TILE
    mesh = plsc.VectorSubcoreMesh(core_axis_name="c", subcore_axis_name="s")

    c1_total = int(round(total_chunks * SC1_SHARE / 8)) * 8
    c0_total = total_chunks - c1_total
    assert c0_total % 8 == 0
    b0 = c0_total // 16 // 8 * 8
    r0 = (c0_total - 16 * b0) // 8
    b1 = c1_total // 16 // 8 * 8
    r1 = (c1_total - 16 * b1) // 8
    assert b1 >= 8 and r0 <= 16 and r1 <= 16

    @functools.partial(
        pl.kernel,
        mesh=mesh,
        out_type=jax.ShapeDtypeStruct((N_SC, n_pad, d), jnp.float32),
        scratch_types=[
            pltpu.VMEM((8 * CHUNK,), jnp.int32),
            pltpu.VMEM((8 * CHUNK,), jnp.int32),
            pltpu.VMEM((4, CHUNK), jnp.int32),
            pltpu.VMEM((4, CHUNK, d), jnp.float32),
            pltpu.VMEM_SHARED((n_pad, d), jnp.float32),
            pltpu.SemaphoreType.DMA,
            pltpu.SemaphoreType.DMA,
            pltpu.SemaphoreType.DMA,
            pltpu.SemaphoreType.DMA,
            pltpu.SemaphoreType.DMA,
            pltpu.SemaphoreType.DMA,
            pltpu.SemaphoreType.DMA,
            pltpu.SemaphoreType.DMA,
        ],
    )
    def seg(x_hbm, ei_hbm, out_hbm, sring, dring, wring, buf, acc,
            semg_a, semg_b, sems_a, sems_b, semi0, semi1, semi2, semi3):
        c = lax.axis_index("c")
        s = lax.axis_index("s")
        is0 = c == 0
        nch = jnp.where(is0, b0 + 8 * (s < r0), b1 + 8 * (s < r1))
        off = jnp.where(is0, b0 * s + 8 * jnp.minimum(s, r0),
                        c0_total + b1 * s + 8 * jnp.minimum(s, r1))
        semi = [semi0, semi1, semi2, semi3]

        def load_idx(gbase, slot):
            e0 = (off + gbase) * CHUNK
            pltpu.async_copy(ei_hbm.at[pl.ds(e0, 2 * CHUNK)],
                             sring.at[pl.ds(slot * 2 * CHUNK, 2 * CHUNK)],
                             semi[slot])
            pltpu.async_copy(ei_hbm.at[pl.ds(n_edges + e0, 2 * CHUNK)],
                             dring.at[pl.ds(slot * 2 * CHUNK, 2 * CHUNK)],
                             semi[slot])

        def wait_idx(gbase, slot):
            e0 = (off + gbase) * CHUNK
            pltpu.make_async_copy(
                ei_hbm.at[pl.ds(e0, 2 * CHUNK)],
                sring.at[pl.ds(slot * 2 * CHUNK, 2 * CHUNK)],
                semi[slot]).wait()
            pltpu.make_async_copy(
                ei_hbm.at[pl.ds(n_edges + e0, 2 * CHUNK)],
                dring.at[pl.ds(slot * 2 * CHUNK, 2 * CHUNK)],
                semi[slot]).wait()

        def dstcopy(slot, row0):
            for q in (0, 1):
                for k in range(0, CHUNK, 16):
                    wring[row0 + q, pl.ds(k, 16)] = dring[
                        pl.ds((2 * slot + q) * CHUNK + k, 16)]

        def gather(slot, dslot0, sem):
            for q in (0, 1):
                pltpu.async_copy(
                    x_hbm.at[sring.at[pl.ds((2 * slot + q) * CHUNK, CHUNK)]],
                    buf.at[dslot0 + q], sem)

        def gather_wait(slot, dslot0, sem):
            for q in (0, 1):
                pltpu.make_async_copy(
                    x_hbm.at[sring.at[pl.ds((2 * slot + q) * CHUNK, CHUNK)]],
                    buf.at[dslot0 + q], sem).wait()

        def scat(dslot0, sem):
            for q in (0, 1):
                pltpu.async_copy(buf.at[dslot0 + q],
                                 acc.at[wring.at[dslot0 + q]], sem, add=True)

        def scat_wait(dslot0, sem):
            for q in (0, 1):
                pltpu.make_async_copy(buf.at[dslot0 + q],
                                      acc.at[wring.at[dslot0 + q]],
                                      sem).wait()

        for g in range(4):
            load_idx(2 * g, g)

        @pl.loop(0, CHUNK)
        def _(i):
            @pl.loop(0, d, step=16)
            def _(j):
                buf[0, i, pl.ds(j, 16)] = jnp.zeros((16,), jnp.float32)

        base = s * rows_per_tile
        whole = rows_per_tile // CHUNK * CHUNK

        @pl.loop(0, whole, step=CHUNK)
        def _(r):
            pltpu.sync_copy(buf.at[0], acc.at[pl.ds(base + r, CHUNK)])

        if rows_per_tile > whole:
            rem = rows_per_tile - whole
            pltpu.sync_copy(buf.at[0, pl.ds(0, rem)],
                            acc.at[pl.ds(base + whole, rem)])

        plsc.subcore_barrier()

        wait_idx(0, 0)
        gather(0, 0, semg_a)

        @pl.loop(0, nch, step=8)
        def _(j):
            gsems = (semg_a, semg_b)
            ssems = (sems_a, sems_b)
            for p in range(4):
                d0 = 2 * (p % 2)
                nd0 = 2 * ((p + 1) % 2)
                gather_wait(p, d0, gsems[p % 2])
                dstcopy(p, d0)
                scat(d0, ssems[p % 2])
                if p == 0:
                    @pl.when(j > 0)
                    def _():
                        scat_wait(2, ssems[1])
                else:
                    scat_wait(nd0, ssems[(p + 1) % 2])

                @pl.when(j + 8 + 2 * p < nch)
                def _():
                    load_idx(j + 8 + 2 * p, p)

                if p < 3:
                    wait_idx(j + 2 * (p + 1), p + 1)
                    gather(p + 1, nd0, gsems[(p + 1) % 2])
                else:
                    @pl.when(j + 8 < nch)
                    def _():
                        wait_idx(j + 8, 0)
                        gather(0, 0, gsems[0])

        scat_wait(2, sems_b)

        plsc.subcore_barrier()

        pltpu.sync_copy(
            acc.at[pl.ds(base, rows_per_tile)],
            out_hbm.at[c, pl.ds(base, rows_per_tile)],
        )

    return seg


def _dense(p, xin, w_rel_t, w_root_t, scale, shift, relu):
    n, d = xin.shape
    n_pad = p.shape[1]
    blk = 1024
    grid = (-(-n_pad // blk),)

    def body(p_ref, x_ref, wr_ref, wo_ref, sc_ref, sh_ref, o_ref):
        agg = (p_ref[0] + p_ref[1]).astype(jnp.bfloat16)
        acc = jnp.dot(agg, wr_ref[...].astype(jnp.bfloat16),
                      preferred_element_type=jnp.float32)
        acc += jnp.dot(x_ref[...].astype(jnp.bfloat16),
                       wo_ref[...].astype(jnp.bfloat16),
                       preferred_element_type=jnp.float32)
        h = acc * sc_ref[...] + sh_ref[...]
        if relu:
            h = jnp.maximum(h, 0.0)
        o_ref[...] = h

    return pl.pallas_call(
        body,
        grid=grid,
        in_specs=[
            pl.BlockSpec((2, blk, d), lambda i: (0, i, 0)),
            pl.BlockSpec((blk, d), lambda i: (i, 0)),
            pl.BlockSpec((d, d), lambda i: (0, 0)),
            pl.BlockSpec((d, d), lambda i: (0, 0)),
            pl.BlockSpec((1, d), lambda i: (0, 0)),
            pl.BlockSpec((1, d), lambda i: (0, 0)),
        ],
        out_specs=pl.BlockSpec((blk, d), lambda i: (i, 0)),
        out_shape=jax.ShapeDtypeStruct((n, d), jnp.float32),
    )(p, xin, w_rel_t, w_root_t, scale, shift)


def kernel(x, last_update, edge_index, t, msg, W1_rel, b1_rel, W1_root,
           bn_gamma, bn_beta, bn_mean, bn_var, W2_rel, b2_rel, W2_root):
    n, d = x.shape
    e = edge_index.shape[1]
    assert e % CHUNK == 0

    n_pad = -(-n // (N_TILE * 8)) * (N_TILE * 8)

    seg = _segment_sum_partials(n, d, n_pad, e // CHUNK)

    s1 = bn_gamma * lax.rsqrt(bn_var + EPS)
    sh1 = (b1_rel - bn_mean) * s1 + bn_beta
    ones = jnp.ones((d,), jnp.float32)

    ei_flat = edge_index.reshape(-1)
    p1 = seg(x, ei_flat)
    h = _dense(p1, x, W1_rel.T, W1_root.T,
               s1.reshape(1, d), sh1.reshape(1, d), relu=True)
    p2 = seg(h, ei_flat)
    out = _dense(p2, h, W2_rel.T, W2_root.T,
                 ones.reshape(1, d), b2_rel.reshape(1, d), relu=False)
    return out
